# baseline (speedup 1.0000x reference)
import functools

import jax
import jax.numpy as jnp
from jax import lax
from jax.experimental import pallas as pl
from jax.experimental.pallas import tpu as pltpu
from jax.experimental.pallas import tpu_sc as plsc

_N = 10000
_E = 160000
_D = 256
_G = 64
_HD = 128
_NC = 2
_NS = 16
_CH = 127
_EPT = _E // _NS
_NCH = 80
_NCHH = _NCH // 2
_EPT_PAD = _NCH * _CH
_ACC_ROWS = 10240
_BN = 2000
_NB = _N // _BN
_OP = 128


def _segsum_body(yp, srcs, dsts, out, src_v, dst_v, rows0, rows1, acc,
                 sem0, sem1):
    c = lax.axis_index("c")
    s = lax.axis_index("s")

    z16 = jnp.zeros((16,), jnp.float32)

    @pl.loop(0, 64)
    def _zrow(r):
        for k in range(_HD // 16):
            rows0[r, pl.ds(k * 16, 16)] = z16

    rows_per_tile = _ACC_ROWS // _NS

    @pl.loop(0, rows_per_tile // 64)
    def _zacc(i):
        pltpu.sync_copy(rows0.at[pl.ds(0, 64)],
                        acc.at[pl.ds(s * rows_per_tile + i * 64, 64)])

    plsc.subcore_barrier()

    for half in range(2):
        pltpu.sync_copy(srcs.at[c, s, pl.ds(half * _NCHH, _NCHH)], src_v)
        pltpu.sync_copy(dsts.at[s, pl.ds(half * _NCHH, _NCHH)], dst_v)
        pltpu.async_copy(yp.at[src_v.at[0]], rows0, sem0)

        @pl.loop(0, _NCHH // 2 - 1)
        def _chunk(jo):
            j = jo * 2
            pltpu.make_async_copy(yp.at[src_v.at[j]], rows0, sem0).wait()
            pltpu.async_copy(yp.at[src_v.at[j + 1]], rows1, sem1)
            pltpu.sync_copy(rows0, acc.at[dst_v.at[j]], add=True)
            pltpu.make_async_copy(yp.at[src_v.at[j + 1]], rows1, sem1).wait()
            pltpu.async_copy(yp.at[src_v.at[j + 2]], rows0, sem0)
            pltpu.sync_copy(rows1, acc.at[dst_v.at[j + 1]], add=True)

        jl = _NCHH - 2
        pltpu.make_async_copy(yp.at[src_v.at[jl]], rows0, sem0).wait()
        pltpu.async_copy(yp.at[src_v.at[jl + 1]], rows1, sem1)
        pltpu.sync_copy(rows0, acc.at[dst_v.at[jl]], add=True)
        pltpu.make_async_copy(yp.at[src_v.at[jl + 1]], rows1, sem1).wait()
        pltpu.sync_copy(rows1, acc.at[dst_v.at[jl + 1]], add=True)

    plsc.subcore_barrier()

    @pl.when(s < _NS - 1)
    def _wb():
        pltpu.sync_copy(acc.at[pl.ds(s * 624, 624)],
                        out.at[pl.ds(c * _N + s * 624, 624)])

    @pl.when(s == _NS - 1)
    def _wb_last():
        pltpu.sync_copy(acc.at[pl.ds((_NS - 1) * 624, _N - (_NS - 1) * 624)],
                        out.at[pl.ds(c * _N + (_NS - 1) * 624,
                                     _N - (_NS - 1) * 624)])


@functools.cache
def _get_segsum():
    return pl.kernel(
        _segsum_body,
        out_type=jax.ShapeDtypeStruct((2 * _N, _HD), jnp.float32),
        mesh=plsc.VectorSubcoreMesh(core_axis_name="c", subcore_axis_name="s"),
        scratch_types=[
            pltpu.VMEM((_NCHH, _CH), jnp.int32),
            pltpu.VMEM((_NCHH, _CH), jnp.int32),
            pltpu.VMEM((_CH, _HD), jnp.float32),
            pltpu.VMEM((_CH, _HD), jnp.float32),
            pltpu.VMEM_SHARED((_ACC_ROWS, _HD), jnp.float32),
            pltpu.SemaphoreType.DMA,
            pltpu.SemaphoreType.DMA,
        ],
    )


_DN = (((1,), (1,)), ((), ()))


def _l1_body(x_r, wrel_r, wroot_r, b_r, y_r, r_r):
    xb = x_r[...]
    y_r[...] = lax.dot_general(xb, wrel_r[...], _DN,
                               preferred_element_type=jnp.float32)
    r_r[...] = lax.dot_general(xb, wroot_r[...], _DN,
                               preferred_element_type=jnp.float32) + b_r[0]


_l1 = pl.pallas_call(
    _l1_body,
    grid=(_NB, _NC),
    in_specs=[
        pl.BlockSpec((_BN, _D), lambda n, c: (n, 0)),
        pl.BlockSpec((_HD, _D), lambda n, c: (c, 0)),
        pl.BlockSpec((_HD, _D), lambda n, c: (c, 0)),
        pl.BlockSpec((1, 1, _HD), lambda n, c: (c, 0, 0)),
    ],
    out_specs=[
        pl.BlockSpec((_BN, _HD), lambda n, c: (c * _NB + n, 0)),
        pl.BlockSpec((_BN, _HD), lambda n, c: (n, c)),
    ],
    out_shape=[
        jax.ShapeDtypeStruct((2 * _N, _HD), jnp.float32),
        jax.ShapeDtypeStruct((_N, _D), jnp.float32),
    ],
)


def _l2_body(lo_r, hi_r, rin_r, wrel_r, wroot_r, b_r, y_r, r_r):
    agg = jnp.concatenate([lo_r[...], hi_r[...]], axis=1)
    h = jnp.maximum(agg + rin_r[...], 0.0)
    y_r[...] = lax.dot_general(h, wrel_r[...], _DN,
                               preferred_element_type=jnp.float32)
    r_r[...] = lax.dot_general(h, wroot_r[...], _DN,
                               preferred_element_type=jnp.float32) + b_r[0]


_l2 = pl.pallas_call(
    _l2_body,
    grid=(_NB, _NC),
    in_specs=[
        pl.BlockSpec((_BN, _HD), lambda n, c: (n, 0)),
        pl.BlockSpec((_BN, _HD), lambda n, c: (_NB + n, 0)),
        pl.BlockSpec((_BN, _D), lambda n, c: (n, 0)),
        pl.BlockSpec((_HD, _D), lambda n, c: (c, 0)),
        pl.BlockSpec((_HD, _D), lambda n, c: (c, 0)),
        pl.BlockSpec((1, 1, _HD), lambda n, c: (c, 0, 0)),
    ],
    out_specs=[
        pl.BlockSpec((_BN, _HD), lambda n, c: (c * _NB + n, 0)),
        pl.BlockSpec((_BN, _HD), lambda n, c: (n, c)),
    ],
    out_shape=[
        jax.ShapeDtypeStruct((2 * _N, _HD), jnp.float32),
        jax.ShapeDtypeStruct((_N, _D), jnp.float32),
    ],
)


def _head_body(lo_r, hi_r, rin_r, batch_r, wfc1_r, bfc1_r, wfc2_r, bfc2_r,
               out_r, acc_r, cnt_r):
    n = pl.program_id(0)

    @pl.when(n == 0)
    def _():
        acc_r[...] = jnp.zeros_like(acc_r)
        cnt_r[...] = jnp.zeros_like(cnt_r)

    agg = jnp.concatenate([lo_r[...], hi_r[...]], axis=1)
    h2 = jnp.maximum(agg + rin_r[...], 0.0)
    b = batch_r[0, 0, :]
    gids = lax.broadcasted_iota(jnp.int32, (_G, _BN), 0)
    mask = (gids == b[None, :]).astype(jnp.float32)
    acc_r[...] += lax.dot_general(mask, h2, (((1,), (0,)), ((), ())),
                                  preferred_element_type=jnp.float32)
    cnt_r[...] += jnp.broadcast_to(
        jnp.sum(mask, axis=1, keepdims=True), cnt_r.shape)

    @pl.when(n == _NB - 1)
    def _():
        pooled = acc_r[...] / jnp.clip(cnt_r[:, 0:1], 1.0, None)
        t = jnp.maximum(
            lax.dot_general(pooled, wfc1_r[...], _DN,
                            preferred_element_type=jnp.float32) + bfc1_r[...],
            0.0)
        out_r[...] = lax.dot_general(t, wfc2_r[...], _DN,
                                     preferred_element_type=jnp.float32
                                     ) + bfc2_r[...]


_head = pl.pallas_call(
    _head_body,
    grid=(_NB,),
    in_specs=[
        pl.BlockSpec((_BN, _HD), lambda n: (n, 0)),
        pl.BlockSpec((_BN, _HD), lambda n: (_NB + n, 0)),
        pl.BlockSpec((_BN, _D), lambda n: (n, 0)),
        pl.BlockSpec((1, 1, _BN), lambda n: (n, 0, 0)),
        pl.BlockSpec((_D, _D), lambda n: (0, 0)),
        pl.BlockSpec((1, _D), lambda n: (0, 0)),
        pl.BlockSpec((_OP, _D), lambda n: (0, 0)),
        pl.BlockSpec((1, _OP), lambda n: (0, 0)),
    ],
    out_specs=pl.BlockSpec((_G, _OP), lambda n: (0, 0)),
    out_shape=jax.ShapeDtypeStruct((_G, _OP), jnp.float32),
    scratch_shapes=[
        pltpu.VMEM((_G, _D), jnp.float32),
        pltpu.VMEM((_G, _HD), jnp.float32),
    ],
)


def kernel(x, edge_index, batch, W1_rel, b1_rel, W1_root, W2_rel, b2_rel,
           W2_root, W_fc1, b_fc1, W_fc2, b_fc2):
    ei = edge_index.astype(jnp.int32)
    bt = batch.astype(jnp.int32)
    src = ei[0].reshape(_NS, _EPT)
    dst = ei[1].reshape(_NS, _EPT)
    pad = _EPT_PAD - _EPT
    src_t = jnp.pad(src, ((0, 0), (0, pad))).reshape(_NS, _NCH, _CH)
    dst_t = jnp.pad(dst, ((0, 0), (0, pad)),
                    constant_values=_N).reshape(_NS, _NCH, _CH)
    src2 = jnp.stack([src_t, src_t + _N])

    b1_3 = b1_rel.reshape(_NC, 1, _HD)
    b2_3 = b2_rel.reshape(_NC, 1, _HD)
    batch3 = bt.reshape(_NB, 1, _BN)
    wfc2p = jnp.pad(W_fc2, ((0, _OP - W_fc2.shape[0]), (0, 0)))
    bfc2p = jnp.pad(b_fc2, (0, _OP - b_fc2.shape[0])).reshape(1, _OP)
    bfc1 = b_fc1.reshape(1, _D)

    segsum = _get_segsum()
    y1p, r1 = _l1(x, W1_rel, W1_root, b1_3)
    agg1 = segsum(y1p, src2, dst_t)
    y2p, r2 = _l2(agg1, agg1, r1, W2_rel, W2_root, b2_3)
    agg2 = segsum(y2p, src2, dst_t)
    outp = _head(agg2, agg2, r2, batch3, W_fc1, bfc1, wfc2p, bfc2p)
    return outp[:, :10]


# speedup vs baseline: 1.5510x; 1.5510x over previous
import functools

import jax
import jax.numpy as jnp
from jax import lax
from jax.experimental import pallas as pl
from jax.experimental.pallas import tpu as pltpu
from jax.experimental.pallas import tpu_sc as plsc

_N = 10000
_E = 160000
_D = 256
_G = 64
_HD = 128
_NC = 2
_NS = 16
_CH = 125
_EPT = _E // _NS
_NCH = 80
_NCHH = _NCH // 2
_EPT_PAD = _NCH * _CH
_ACC_ROWS = 10240
_BN = 2000
_NB = _N // _BN
_OP = 128


def _segsum_body(yp, srcs, dsts, out, src_v, dst_v, rows0, rows1, acc,
                 sem0, sem1):
    c = lax.axis_index("c")
    s = lax.axis_index("s")

    z16 = jnp.zeros((16,), jnp.float32)

    @pl.loop(0, 64)
    def _zrow(r):
        for k in range(_HD // 16):
            rows0[r, pl.ds(k * 16, 16)] = z16

    rows_per_tile = _ACC_ROWS // _NS

    @pl.loop(0, rows_per_tile // 64)
    def _zacc(i):
        pltpu.sync_copy(rows0.at[pl.ds(0, 64)],
                        acc.at[pl.ds(s * rows_per_tile + i * 64, 64)])

    plsc.subcore_barrier()

    for half in range(2):
        pltpu.sync_copy(srcs.at[c, s, pl.ds(half * _NCHH, _NCHH)], src_v)
        pltpu.sync_copy(dsts.at[s, pl.ds(half * _NCHH, _NCHH)], dst_v)
        pltpu.async_copy(yp.at[src_v.at[0]], rows0, sem0)

        @pl.loop(0, _NCHH // 2 - 1)
        def _chunk(jo):
            j = jo * 2
            pltpu.make_async_copy(yp.at[src_v.at[j]], rows0, sem0).wait()
            pltpu.async_copy(yp.at[src_v.at[j + 1]], rows1, sem1)
            pltpu.sync_copy(rows0, acc.at[dst_v.at[j]], add=True)
            pltpu.make_async_copy(yp.at[src_v.at[j + 1]], rows1, sem1).wait()
            pltpu.async_copy(yp.at[src_v.at[j + 2]], rows0, sem0)
            pltpu.sync_copy(rows1, acc.at[dst_v.at[j + 1]], add=True)

        jl = _NCHH - 2
        pltpu.make_async_copy(yp.at[src_v.at[jl]], rows0, sem0).wait()
        pltpu.async_copy(yp.at[src_v.at[jl + 1]], rows1, sem1)
        pltpu.sync_copy(rows0, acc.at[dst_v.at[jl]], add=True)
        pltpu.make_async_copy(yp.at[src_v.at[jl + 1]], rows1, sem1).wait()
        pltpu.sync_copy(rows1, acc.at[dst_v.at[jl + 1]], add=True)

    plsc.subcore_barrier()

    @pl.when(s < _NS - 1)
    def _wb():
        pltpu.sync_copy(acc.at[pl.ds(s * 624, 624)],
                        out.at[pl.ds(c * _N + s * 624, 624)])

    @pl.when(s == _NS - 1)
    def _wb_last():
        pltpu.sync_copy(acc.at[pl.ds((_NS - 1) * 624, _N - (_NS - 1) * 624)],
                        out.at[pl.ds(c * _N + (_NS - 1) * 624,
                                     _N - (_NS - 1) * 624)])


@functools.cache
def _get_segsum():
    return pl.kernel(
        _segsum_body,
        out_type=jax.ShapeDtypeStruct((2 * _N, _HD), jnp.float32),
        mesh=plsc.VectorSubcoreMesh(core_axis_name="c", subcore_axis_name="s"),
        scratch_types=[
            pltpu.VMEM((_NCHH, _CH), jnp.int32),
            pltpu.VMEM((_NCHH, _CH), jnp.int32),
            pltpu.VMEM((_CH, _HD), jnp.float32),
            pltpu.VMEM((_CH, _HD), jnp.float32),
            pltpu.VMEM_SHARED((_ACC_ROWS, _HD), jnp.float32),
            pltpu.SemaphoreType.DMA,
            pltpu.SemaphoreType.DMA,
        ],
    )


_DN = (((1,), (1,)), ((), ()))


def _l1_body(x_r, wrel_r, wroot_r, b_r, y_r, r_r):
    xb = x_r[...]
    y_r[...] = lax.dot_general(xb, wrel_r[...], _DN,
                               preferred_element_type=jnp.float32)
    r_r[...] = lax.dot_general(xb, wroot_r[...], _DN,
                               preferred_element_type=jnp.float32) + b_r[0]


_l1 = pl.pallas_call(
    _l1_body,
    grid=(_NB, _NC),
    in_specs=[
        pl.BlockSpec((_BN, _D), lambda n, c: (n, 0)),
        pl.BlockSpec((_HD, _D), lambda n, c: (c, 0)),
        pl.BlockSpec((_HD, _D), lambda n, c: (c, 0)),
        pl.BlockSpec((1, 1, _HD), lambda n, c: (c, 0, 0)),
    ],
    out_specs=[
        pl.BlockSpec((_BN, _HD), lambda n, c: (c * _NB + n, 0)),
        pl.BlockSpec((_BN, _HD), lambda n, c: (n, c)),
    ],
    out_shape=[
        jax.ShapeDtypeStruct((2 * _N, _HD), jnp.float32),
        jax.ShapeDtypeStruct((_N, _D), jnp.float32),
    ],
)


def _l2_body(lo_r, hi_r, rin_r, wrel_r, wroot_r, b_r, y_r, r_r):
    agg = jnp.concatenate([lo_r[...], hi_r[...]], axis=1)
    h = jnp.maximum(agg + rin_r[...], 0.0)
    y_r[...] = lax.dot_general(h, wrel_r[...], _DN,
                               preferred_element_type=jnp.float32)
    r_r[...] = lax.dot_general(h, wroot_r[...], _DN,
                               preferred_element_type=jnp.float32) + b_r[0]


_l2 = pl.pallas_call(
    _l2_body,
    grid=(_NB, _NC),
    in_specs=[
        pl.BlockSpec((_BN, _HD), lambda n, c: (n, 0)),
        pl.BlockSpec((_BN, _HD), lambda n, c: (_NB + n, 0)),
        pl.BlockSpec((_BN, _D), lambda n, c: (n, 0)),
        pl.BlockSpec((_HD, _D), lambda n, c: (c, 0)),
        pl.BlockSpec((_HD, _D), lambda n, c: (c, 0)),
        pl.BlockSpec((1, 1, _HD), lambda n, c: (c, 0, 0)),
    ],
    out_specs=[
        pl.BlockSpec((_BN, _HD), lambda n, c: (c * _NB + n, 0)),
        pl.BlockSpec((_BN, _HD), lambda n, c: (n, c)),
    ],
    out_shape=[
        jax.ShapeDtypeStruct((2 * _N, _HD), jnp.float32),
        jax.ShapeDtypeStruct((_N, _D), jnp.float32),
    ],
)


def _head_body(lo_r, hi_r, rin_r, batch_r, wfc1_r, bfc1_r, wfc2_r, bfc2_r,
               out_r, acc_r, cnt_r):
    n = pl.program_id(0)

    @pl.when(n == 0)
    def _():
        acc_r[...] = jnp.zeros_like(acc_r)
        cnt_r[...] = jnp.zeros_like(cnt_r)

    agg = jnp.concatenate([lo_r[...], hi_r[...]], axis=1)
    h2 = jnp.maximum(agg + rin_r[...], 0.0)
    b = batch_r[0, 0, :]
    gids = lax.broadcasted_iota(jnp.int32, (_G, _BN), 0)
    mask = (gids == b[None, :]).astype(jnp.float32)
    acc_r[...] += lax.dot_general(mask, h2, (((1,), (0,)), ((), ())),
                                  preferred_element_type=jnp.float32)
    cnt_r[...] += jnp.broadcast_to(
        jnp.sum(mask, axis=1, keepdims=True), cnt_r.shape)

    @pl.when(n == _NB - 1)
    def _():
        pooled = acc_r[...] / jnp.clip(cnt_r[:, 0:1], 1.0, None)
        t = jnp.maximum(
            lax.dot_general(pooled, wfc1_r[...], _DN,
                            preferred_element_type=jnp.float32) + bfc1_r[...],
            0.0)
        out_r[...] = lax.dot_general(t, wfc2_r[...], _DN,
                                     preferred_element_type=jnp.float32
                                     ) + bfc2_r[...]


_head = pl.pallas_call(
    _head_body,
    grid=(_NB,),
    in_specs=[
        pl.BlockSpec((_BN, _HD), lambda n: (n, 0)),
        pl.BlockSpec((_BN, _HD), lambda n: (_NB + n, 0)),
        pl.BlockSpec((_BN, _D), lambda n: (n, 0)),
        pl.BlockSpec((1, 1, _BN), lambda n: (n, 0, 0)),
        pl.BlockSpec((_D, _D), lambda n: (0, 0)),
        pl.BlockSpec((1, _D), lambda n: (0, 0)),
        pl.BlockSpec((_OP, _D), lambda n: (0, 0)),
        pl.BlockSpec((1, _OP), lambda n: (0, 0)),
    ],
    out_specs=pl.BlockSpec((_G, _OP), lambda n: (0, 0)),
    out_shape=jax.ShapeDtypeStruct((_G, _OP), jnp.float32),
    scratch_shapes=[
        pltpu.VMEM((_G, _D), jnp.float32),
        pltpu.VMEM((_G, _HD), jnp.float32),
    ],
)


def kernel(x, edge_index, batch, W1_rel, b1_rel, W1_root, W2_rel, b2_rel,
           W2_root, W_fc1, b_fc1, W_fc2, b_fc2):
    ei = edge_index.astype(jnp.int32)
    bt = batch.astype(jnp.int32)
    src = ei[0].reshape(_NS, _EPT)
    dst = ei[1].reshape(_NS, _EPT)
    pad = _EPT_PAD - _EPT
    src_t = jnp.pad(src, ((0, 0), (0, pad))).reshape(_NS, _NCH, _CH)
    dst_t = jnp.pad(dst, ((0, 0), (0, pad)),
                    constant_values=_N).reshape(_NS, _NCH, _CH)
    src2 = jnp.stack([src_t, src_t + _N])

    b1_3 = b1_rel.reshape(_NC, 1, _HD)
    b2_3 = b2_rel.reshape(_NC, 1, _HD)
    batch3 = bt.reshape(_NB, 1, _BN)
    wfc2p = jnp.pad(W_fc2, ((0, _OP - W_fc2.shape[0]), (0, 0)))
    bfc2p = jnp.pad(b_fc2, (0, _OP - b_fc2.shape[0])).reshape(1, _OP)
    bfc1 = b_fc1.reshape(1, _D)

    segsum = _get_segsum()
    y1p, r1 = _l1(x, W1_rel, W1_root, b1_3)
    agg1 = segsum(y1p, src2, dst_t)
    y2p, r2 = _l2(agg1, agg1, r1, W2_rel, W2_root, b2_3)
    agg2 = segsum(y2p, src2, dst_t)
    outp = _head(agg2, agg2, r2, batch3, W_fc1, bfc1, wfc2p, bfc2p)
    return outp[:, :10]


# speedup vs baseline: 1.5640x; 1.0084x over previous
import functools

import jax
import jax.numpy as jnp
from jax import lax
from jax.experimental import pallas as pl
from jax.experimental.pallas import tpu as pltpu
from jax.experimental.pallas import tpu_sc as plsc

_N = 10000
_E = 160000
_D = 256
_G = 64
_HD = 128
_NC = 2
_NS = 16
_CH = 125
_EPT = _E // _NS
_NCH = 80
_NCHH = _NCH // 2
_EPT_PAD = _NCH * _CH
_ACC_ROWS = 10240
_BN = 2000
_NB = _N // _BN
_OP = 128


def _segsum_body(yp, srcs, dsts, out, src_v, dst_v, rows0, rows1, acc,
                 sem0, sem1):
    c = lax.axis_index("c")
    s = lax.axis_index("s")

    z16 = jnp.zeros((16,), jnp.float32)

    @pl.loop(0, 64)
    def _zrow(r):
        for k in range(_HD // 16):
            rows0[r, pl.ds(k * 16, 16)] = z16

    rows_per_tile = _ACC_ROWS // _NS

    @pl.loop(0, rows_per_tile // 64)
    def _zacc(i):
        pltpu.sync_copy(rows0.at[pl.ds(0, 64)],
                        acc.at[pl.ds(s * rows_per_tile + i * 64, 64)])

    plsc.subcore_barrier()

    for half in range(2):
        pltpu.sync_copy(srcs.at[c, s, pl.ds(half * _NCHH, _NCHH)], src_v)
        pltpu.sync_copy(dsts.at[s, pl.ds(half * _NCHH, _NCHH)], dst_v)
        pltpu.async_copy(yp.at[src_v.at[0]], rows0, sem0)

        @pl.loop(0, _NCHH // 2 - 1)
        def _chunk(jo):
            j = jo * 2
            pltpu.make_async_copy(yp.at[src_v.at[j]], rows0, sem0).wait()
            pltpu.async_copy(yp.at[src_v.at[j + 1]], rows1, sem1)
            pltpu.sync_copy(rows0, acc.at[dst_v.at[j]], add=True)
            pltpu.make_async_copy(yp.at[src_v.at[j + 1]], rows1, sem1).wait()
            pltpu.async_copy(yp.at[src_v.at[j + 2]], rows0, sem0)
            pltpu.sync_copy(rows1, acc.at[dst_v.at[j + 1]], add=True)

        jl = _NCHH - 2
        pltpu.make_async_copy(yp.at[src_v.at[jl]], rows0, sem0).wait()
        pltpu.async_copy(yp.at[src_v.at[jl + 1]], rows1, sem1)
        pltpu.sync_copy(rows0, acc.at[dst_v.at[jl]], add=True)
        pltpu.make_async_copy(yp.at[src_v.at[jl + 1]], rows1, sem1).wait()
        pltpu.sync_copy(rows1, acc.at[dst_v.at[jl + 1]], add=True)

    plsc.subcore_barrier()

    @pl.when(s < _NS - 1)
    def _wb():
        pltpu.sync_copy(acc.at[pl.ds(s * 624, 624)],
                        out.at[pl.ds(c * _N + s * 624, 624)])

    @pl.when(s == _NS - 1)
    def _wb_last():
        pltpu.sync_copy(acc.at[pl.ds((_NS - 1) * 624, _N - (_NS - 1) * 624)],
                        out.at[pl.ds(c * _N + (_NS - 1) * 624,
                                     _N - (_NS - 1) * 624)])


@functools.cache
def _get_segsum():
    return pl.kernel(
        _segsum_body,
        out_type=jax.ShapeDtypeStruct((2 * _N, _HD), jnp.float32),
        mesh=plsc.VectorSubcoreMesh(core_axis_name="c", subcore_axis_name="s"),
        scratch_types=[
            pltpu.VMEM((_NCHH, _CH), jnp.int32),
            pltpu.VMEM((_NCHH, _CH), jnp.int32),
            pltpu.VMEM((_CH, _HD), jnp.float32),
            pltpu.VMEM((_CH, _HD), jnp.float32),
            pltpu.VMEM_SHARED((_ACC_ROWS, _HD), jnp.float32),
            pltpu.SemaphoreType.DMA,
            pltpu.SemaphoreType.DMA,
        ],
    )


_DN = (((1,), (1,)), ((), ()))


def _l1a_body(x_r, wrel_r, y_r):
    y_r[...] = lax.dot_general(x_r[...], wrel_r[...], _DN,
                               preferred_element_type=jnp.float32)


_l1a = pl.pallas_call(
    _l1a_body,
    grid=(_NB, _NC),
    in_specs=[
        pl.BlockSpec((_BN, _D), lambda n, c: (n, 0)),
        pl.BlockSpec((_HD, _D), lambda n, c: (c, 0)),
    ],
    out_specs=pl.BlockSpec((_BN, _HD), lambda n, c: (c * _NB + n, 0)),
    out_shape=jax.ShapeDtypeStruct((2 * _N, _HD), jnp.float32),
)


def _l1b_body(x_r, wroot_r, b_r, r_r):
    r_r[...] = lax.dot_general(x_r[...], wroot_r[...], _DN,
                               preferred_element_type=jnp.float32) + b_r[0]


_l1b = pl.pallas_call(
    _l1b_body,
    grid=(_NB, _NC),
    in_specs=[
        pl.BlockSpec((_BN, _D), lambda n, c: (n, 0)),
        pl.BlockSpec((_HD, _D), lambda n, c: (c, 0)),
        pl.BlockSpec((1, 1, _HD), lambda n, c: (c, 0, 0)),
    ],
    out_specs=pl.BlockSpec((_BN, _HD), lambda n, c: (n, c)),
    out_shape=jax.ShapeDtypeStruct((_N, _D), jnp.float32),
)


def _l2a_body(lo_r, hi_r, rin_r, wrel_r, y_r):
    agg = jnp.concatenate([lo_r[...], hi_r[...]], axis=1)
    h = jnp.maximum(agg + rin_r[...], 0.0)
    y_r[...] = lax.dot_general(h, wrel_r[...], _DN,
                               preferred_element_type=jnp.float32)


_l2a = pl.pallas_call(
    _l2a_body,
    grid=(_NB, _NC),
    in_specs=[
        pl.BlockSpec((_BN, _HD), lambda n, c: (n, 0)),
        pl.BlockSpec((_BN, _HD), lambda n, c: (_NB + n, 0)),
        pl.BlockSpec((_BN, _D), lambda n, c: (n, 0)),
        pl.BlockSpec((_HD, _D), lambda n, c: (c, 0)),
    ],
    out_specs=pl.BlockSpec((_BN, _HD), lambda n, c: (c * _NB + n, 0)),
    out_shape=jax.ShapeDtypeStruct((2 * _N, _HD), jnp.float32),
)


def _l2b_body(lo_r, hi_r, rin_r, wroot_r, b_r, r_r):
    agg = jnp.concatenate([lo_r[...], hi_r[...]], axis=1)
    h = jnp.maximum(agg + rin_r[...], 0.0)
    r_r[...] = lax.dot_general(h, wroot_r[...], _DN,
                               preferred_element_type=jnp.float32) + b_r[0]


_l2b = pl.pallas_call(
    _l2b_body,
    grid=(_NB, _NC),
    in_specs=[
        pl.BlockSpec((_BN, _HD), lambda n, c: (n, 0)),
        pl.BlockSpec((_BN, _HD), lambda n, c: (_NB + n, 0)),
        pl.BlockSpec((_BN, _D), lambda n, c: (n, 0)),
        pl.BlockSpec((_HD, _D), lambda n, c: (c, 0)),
        pl.BlockSpec((1, 1, _HD), lambda n, c: (c, 0, 0)),
    ],
    out_specs=pl.BlockSpec((_BN, _HD), lambda n, c: (n, c)),
    out_shape=jax.ShapeDtypeStruct((_N, _D), jnp.float32),
)


def _head_body(lo_r, hi_r, rin_r, batch_r, wfc1_r, bfc1_r, wfc2_r, bfc2_r,
               out_r, acc_r, cnt_r):
    n = pl.program_id(0)

    @pl.when(n == 0)
    def _():
        acc_r[...] = jnp.zeros_like(acc_r)
        cnt_r[...] = jnp.zeros_like(cnt_r)

    agg = jnp.concatenate([lo_r[...], hi_r[...]], axis=1)
    h2 = jnp.maximum(agg + rin_r[...], 0.0)
    b = batch_r[0, 0, :]
    gids = lax.broadcasted_iota(jnp.int32, (_G, _BN), 0)
    mask = (gids == b[None, :]).astype(jnp.float32)
    acc_r[...] += lax.dot_general(mask, h2, (((1,), (0,)), ((), ())),
                                  preferred_element_type=jnp.float32)
    cnt_r[...] += jnp.broadcast_to(
        jnp.sum(mask, axis=1, keepdims=True), cnt_r.shape)

    @pl.when(n == _NB - 1)
    def _():
        pooled = acc_r[...] / jnp.clip(cnt_r[:, 0:1], 1.0, None)
        t = jnp.maximum(
            lax.dot_general(pooled, wfc1_r[...], _DN,
                            preferred_element_type=jnp.float32) + bfc1_r[...],
            0.0)
        out_r[...] = lax.dot_general(t, wfc2_r[...], _DN,
                                     preferred_element_type=jnp.float32
                                     ) + bfc2_r[...]


_head = pl.pallas_call(
    _head_body,
    grid=(_NB,),
    in_specs=[
        pl.BlockSpec((_BN, _HD), lambda n: (n, 0)),
        pl.BlockSpec((_BN, _HD), lambda n: (_NB + n, 0)),
        pl.BlockSpec((_BN, _D), lambda n: (n, 0)),
        pl.BlockSpec((1, 1, _BN), lambda n: (n, 0, 0)),
        pl.BlockSpec((_D, _D), lambda n: (0, 0)),
        pl.BlockSpec((1, _D), lambda n: (0, 0)),
        pl.BlockSpec((_OP, _D), lambda n: (0, 0)),
        pl.BlockSpec((1, _OP), lambda n: (0, 0)),
    ],
    out_specs=pl.BlockSpec((_G, _OP), lambda n: (0, 0)),
    out_shape=jax.ShapeDtypeStruct((_G, _OP), jnp.float32),
    scratch_shapes=[
        pltpu.VMEM((_G, _D), jnp.float32),
        pltpu.VMEM((_G, _HD), jnp.float32),
    ],
)


def kernel(x, edge_index, batch, W1_rel, b1_rel, W1_root, W2_rel, b2_rel,
           W2_root, W_fc1, b_fc1, W_fc2, b_fc2):
    ei = edge_index.astype(jnp.int32)
    bt = batch.astype(jnp.int32)
    src = ei[0].reshape(_NS, _EPT)
    dst = ei[1].reshape(_NS, _EPT)
    pad = _EPT_PAD - _EPT
    src_t = jnp.pad(src, ((0, 0), (0, pad))).reshape(_NS, _NCH, _CH)
    dst_t = jnp.pad(dst, ((0, 0), (0, pad)),
                    constant_values=_N).reshape(_NS, _NCH, _CH)
    src2 = jnp.stack([src_t, src_t + _N])

    b1_3 = b1_rel.reshape(_NC, 1, _HD)
    b2_3 = b2_rel.reshape(_NC, 1, _HD)
    batch3 = bt.reshape(_NB, 1, _BN)
    wfc2p = jnp.pad(W_fc2, ((0, _OP - W_fc2.shape[0]), (0, 0)))
    bfc2p = jnp.pad(b_fc2, (0, _OP - b_fc2.shape[0])).reshape(1, _OP)
    bfc1 = b_fc1.reshape(1, _D)

    segsum = _get_segsum()
    y1p = _l1a(x, W1_rel)
    r1 = _l1b(x, W1_root, b1_3)
    agg1 = segsum(y1p, src2, dst_t)
    y2p = _l2a(agg1, agg1, r1, W2_rel)
    r2 = _l2b(agg1, agg1, r1, W2_root, b2_3)
    agg2 = segsum(y2p, src2, dst_t)
    outp = _head(agg2, agg2, r2, batch3, W_fc1, bfc1, wfc2p, bfc2p)
    return outp[:, :10]
